# baseline (speedup 1.0000x reference)
import functools

import jax
import jax.numpy as jnp
from jax import lax
from jax.experimental import pallas as pl
from jax.experimental.pallas import tpu as pltpu
from jax.experimental.pallas import tpu_sc as plsc

B = 16384
D = 64
NC = 2
NS = 16
NW = NC * NS

R_U = 1000000
R_M = 100000
COLS_U = 245
COLS_M = 25
TILE_R_U = COLS_U * 128
TILE_R_M = COLS_M * 128
TAIL_U = 999936
TAIL_W_U = R_U - TAIL_U
TAIL_M = 99968
TAIL_W_M = R_M - TAIL_M
CAP = 704
CCAP = 128
NVEC_IDX = B // 16
NSC = CAP // 64
DUMMY = B
OUT_ROWS = B + 8


def _iota16():
    return lax.iota(jnp.int32, 16)


def _scalar(vec):
    return lax.reduce_max(vec, axes=(0,))


def _extract(vec, lane):
    return _scalar(jnp.where(_iota16() == lane, vec, jnp.int32(-2**31)))


def _popcount(mask):
    return _scalar(plsc.all_reduce_population_count(mask))


@functools.lru_cache(maxsize=None)
def _build_gather():
    mesh = plsc.VectorSubcoreMesh(core_axis_name="c", subcore_axis_name="s",
                                  num_cores=NC)

    @functools.partial(
        pl.kernel,
        mesh=mesh,
        out_type=(
            jax.ShapeDtypeStruct((OUT_ROWS, 128), jnp.float32),
            jax.ShapeDtypeStruct((OUT_ROWS, 128), jnp.float32),
        ),
        scratch_types=[
            pltpu.VMEM((B,), jnp.int32),
            pltpu.VMEM((64, 128), jnp.float32),
            pltpu.VMEM((64, 128), jnp.float32),
            pltpu.VMEM((CAP,), jnp.int32),
            pltpu.VMEM((CAP,), jnp.int32),
            pltpu.VMEM((CCAP,), jnp.int32),
            pltpu.VMEM((CCAP,), jnp.int32),
            pltpu.VMEM((CAP, 128), jnp.float32),
            pltpu.VMEM((CAP,), jnp.int32),
            pltpu.VMEM((NSC, 64), jnp.int32),
            pltpu.SemaphoreType.DMA,
            pltpu.SemaphoreType.DMA,
            pltpu.SemaphoreType.DMA,
        ],
        compiler_params=pltpu.CompilerParams(use_tc_tiling_on_sc=True,
                                             needs_layout_passes=False),
    )
    def gather(users_hbm, movies_hbm, ut_hbm, mt_hbm, uout_hbm, mout_hbm,
               idx_v, buf0, buf1, loc_r, loc_j, ch_r, ch_j, st_rows, st_j,
               jj, sem0, sem1, sem_s):
        wid = lax.axis_index("s") * NC + lax.axis_index("c")
        i16 = _iota16()
        big = jnp.full((16,), 2**30, jnp.int32)
        dummy16 = jnp.full((16,), DUMMY, jnp.int32)

        def run_table(idx_hbm, tab_hbm, out_hbm, tile_r, ncols, tail, tail_w):
            base = wid * tile_r
            last_full = tail - 128

            for v in range(CAP // 16):
                loc_r[pl.ds(16 * v, 16)] = big
                st_j[pl.ds(16 * v, 16)] = dummy16

            pltpu.sync_copy(idx_hbm, idx_v)

            def scan_body(v, cur):
                rv = idx_v[pl.ds(16 * v, 16)]
                jv = 16 * v + i16
                m = (rv >= base) & (rv < base + tile_r)
                cur = jnp.minimum(cur, CAP - 16)
                plsc.store_compressed(loc_r.at[pl.ds(cur, 16)], rv, mask=m)
                plsc.store_compressed(loc_j.at[pl.ds(cur, 16)], jv, mask=m)
                return cur + _popcount(m)

            nloc = jnp.int32(0)
            nlvec = (nloc + 15) // 16

            def chunk_lo(k):
                return jnp.minimum(base + 128 * k, last_full)

            def start_dma(k, buf, sem):
                return pltpu.async_copy(
                    tab_hbm.at[pl.ds(0, 64), pl.ds(chunk_lo(k), 128)],
                    buf, sem)

            def wait_dma(k, buf, sem):
                pltpu.make_async_copy(
                    tab_hbm.at[pl.ds(0, 64), pl.ds(chunk_lo(k), 128)],
                    buf, sem).wait()

            def process(buf, lo, width, cur_out):
                return cur_out

            def _unused(buf, lo, width, cur_out):
                def sel_body(v, cc):
                    rv = loc_r[pl.ds(16 * v, 16)]
                    jv = loc_j[pl.ds(16 * v, 16)]
                    m = (rv >= lo) & (rv < lo + width)
                    cc = jnp.minimum(cc, CCAP - 16)
                    plsc.store_compressed(ch_r.at[pl.ds(cc, 16)], rv, mask=m)
                    plsc.store_compressed(ch_j.at[pl.ds(cc, 16)], jv, mask=m)
                    return cc + _popcount(m)

                nh = lax.fori_loop(0, nlvec, sel_body, jnp.int32(0))

                def row_body(i, cur):
                    w0 = (i // 16) * 16
                    li = i - w0
                    rsc = _extract(ch_r[pl.ds(w0, 16)], li)
                    jsc = _extract(ch_j[pl.ds(w0, 16)], li)
                    lane = jnp.broadcast_to(rsc - lo, (16,))
                    pos = jnp.minimum(cur, CAP - 1)
                    for q in range(4):
                        vals = plsc.load_gather(buf, [i16 + 16 * q, lane])
                        st_rows[pos, pl.ds(16 * q, 16)] = vals
                    plsc.store_scatter(st_j, [jnp.broadcast_to(pos, (16,))],
                                       jnp.broadcast_to(jsc, (16,)),
                                       mask=i16 == 0)
                    return cur + 1

                return lax.fori_loop(0, nh, row_body, cur_out)

            cp0 = start_dma(0, buf0, sem0)
            cp1 = start_dma(1, buf1, sem1)
            del cp0, cp1

            def pair_body(g, cur_out):
                k0 = 2 * g
                wait_dma(k0, buf0, sem0)
                cur_out = process(buf0, chunk_lo(k0), 128, cur_out)
                start_dma(k0 + 2, buf0, sem0)
                wait_dma(k0 + 1, buf1, sem1)
                cur_out = process(buf1, chunk_lo(k0 + 1), 128, cur_out)
                start_dma(k0 + 3, buf1, sem1)
                return cur_out

            npair = ncols // 2
            cur_out = lax.fori_loop(0, npair, pair_body, jnp.int32(0))
            wait_dma(2 * npair, buf0, sem0)
            wait_dma(2 * npair + 1, buf1, sem1)
            if ncols % 2:
                cur_out = process(buf0, chunk_lo(2 * npair), 128, cur_out)
            tail_dyn = jnp.int32(tail) + jnp.where(wid >= NW, 128, 0)
            pltpu.sync_copy(tab_hbm.at[pl.ds(0, 64), pl.ds(tail_dyn, 128)],
                            buf1)
            cur_out = process(buf1, jnp.int32(tail), tail_w, cur_out)

            for c in range(NSC):
                for t in range(4):
                    jj[c, pl.ds(16 * t, 16)] = st_j[pl.ds(64 * c + 16 * t, 16)]
            copies = [
                pltpu.async_copy(st_rows.at[pl.ds(64 * c, 64)],
                                 out_hbm.at[jj.at[c]], sem_s)
                for c in range(NSC)
            ]
            for cp in copies:
                cp.wait()

        run_table(users_hbm, ut_hbm, uout_hbm, TILE_R_U, COLS_U, TAIL_U,
                  TAIL_W_U)
        run_table(movies_hbm, mt_hbm, mout_hbm, TILE_R_M, COLS_M, TAIL_M,
                  TAIL_W_M)

    return gather


MBLK = 2048


def _r16(x):
    return x.astype(jnp.bfloat16).astype(jnp.float32)


def _mlp_body(ue, me, w1a, w1b, b1, w2, b2, w3, b3, out):
    h = jnp.dot(_r16(ue[:, :D]), w1a[...], preferred_element_type=jnp.float32,
                precision=lax.Precision.HIGHEST)
    h = h + jnp.dot(_r16(me[:, :D]), w1b[...],
                    preferred_element_type=jnp.float32,
                    precision=lax.Precision.HIGHEST)
    h = _r16(jnp.maximum(h + b1[...], 0.0))
    h = _r16(jnp.maximum(
        jnp.dot(h, w2[...], preferred_element_type=jnp.float32,
                precision=lax.Precision.HIGHEST) + b2[...], 0.0))
    out[...] = jnp.maximum(jnp.sum(h * w3[...], axis=1) + b3[0, 0], 0.0)


def kernel(users, movies, user_table, movie_table, W1, b1, W2, b2, W3, b3):
    uo, mo = _build_gather()(users.astype(jnp.int32),
                             movies.astype(jnp.int32),
                             user_table.T, movie_table.T)
    out = pl.pallas_call(
        _mlp_body,
        grid=(B // MBLK,),
        in_specs=[
            pl.BlockSpec((MBLK, 128), lambda i: (i, 0)),
            pl.BlockSpec((MBLK, 128), lambda i: (i, 0)),
            pl.BlockSpec((D, 64), lambda i: (0, 0)),
            pl.BlockSpec((D, 64), lambda i: (0, 0)),
            pl.BlockSpec((1, 64), lambda i: (0, 0)),
            pl.BlockSpec((64, 16), lambda i: (0, 0)),
            pl.BlockSpec((1, 16), lambda i: (0, 0)),
            pl.BlockSpec((1, 16), lambda i: (0, 0)),
            pl.BlockSpec((1, 1), lambda i: (0, 0)),
        ],
        out_specs=pl.BlockSpec((MBLK,), lambda i: (i,)),
        out_shape=jax.ShapeDtypeStruct((B,), jnp.float32),
    )(uo, mo, W1[:D], W1[D:], b1.reshape(1, 64), W2, b2.reshape(1, 16),
      W3.reshape(1, 16), b3.reshape(1, 1))
    return out


# speedup vs baseline: 1.0016x; 1.0016x over previous
import functools

import jax
import jax.numpy as jnp
from jax import lax
from jax.experimental import pallas as pl
from jax.experimental.pallas import tpu as pltpu
from jax.experimental.pallas import tpu_sc as plsc

B = 16384
D = 64
NC = 2
NS = 16
NW = NC * NS

R_U = 1000000
R_M = 100000
COLS_U = 245
COLS_M = 25
TILE_R_U = COLS_U * 128
TILE_R_M = COLS_M * 128
TAIL_U = 999936
TAIL_W_U = R_U - TAIL_U
TAIL_M = 99968
TAIL_W_M = R_M - TAIL_M
CAP = 704
CCAP = 128
NVEC_IDX = B // 16
NSC = CAP // 64
DUMMY = B
OUT_ROWS = B + 8


def _iota16():
    return lax.iota(jnp.int32, 16)


def _scalar(vec):
    return lax.reduce_max(vec, axes=(0,))


def _extract(vec, lane):
    return _scalar(jnp.where(_iota16() == lane, vec, jnp.int32(-2**31)))


def _popcount(mask):
    return _scalar(plsc.all_reduce_population_count(mask))


@functools.lru_cache(maxsize=None)
def _build_gather():
    mesh = plsc.VectorSubcoreMesh(core_axis_name="c", subcore_axis_name="s",
                                  num_cores=NC)

    @functools.partial(
        pl.kernel,
        mesh=mesh,
        out_type=(
            jax.ShapeDtypeStruct((OUT_ROWS, 128), jnp.float32),
            jax.ShapeDtypeStruct((OUT_ROWS, 128), jnp.float32),
        ),
        scratch_types=[
            pltpu.VMEM((B,), jnp.int32),
            pltpu.VMEM((64, 128), jnp.float32),
            pltpu.VMEM((64, 128), jnp.float32),
            pltpu.VMEM((CAP,), jnp.int32),
            pltpu.VMEM((CAP,), jnp.int32),
            pltpu.VMEM((CCAP,), jnp.int32),
            pltpu.VMEM((CCAP,), jnp.int32),
            pltpu.VMEM((CAP, 128), jnp.float32),
            pltpu.VMEM((CAP,), jnp.int32),
            pltpu.VMEM((NSC, 64), jnp.int32),
            pltpu.SemaphoreType.DMA,
            pltpu.SemaphoreType.DMA,
            pltpu.SemaphoreType.DMA,
        ],
        compiler_params=pltpu.CompilerParams(use_tc_tiling_on_sc=True,
                                             needs_layout_passes=False),
    )
    def gather(users_hbm, movies_hbm, ut_hbm, mt_hbm, uout_hbm, mout_hbm,
               idx_v, buf0, buf1, loc_r, loc_j, ch_r, ch_j, st_rows, st_j,
               jj, sem0, sem1, sem_s):
        wid = lax.axis_index("s") * NC + lax.axis_index("c")
        i16 = _iota16()
        big = jnp.full((16,), 2**30, jnp.int32)
        dummy16 = jnp.full((16,), DUMMY, jnp.int32)

        def run_table(idx_hbm, tab_hbm, out_hbm, tile_r, ncols, tail, tail_w):
            base = wid * tile_r
            last_full = tail - 128

            for v in range(CAP // 16):
                loc_r[pl.ds(16 * v, 16)] = big
                st_j[pl.ds(16 * v, 16)] = dummy16

            pltpu.sync_copy(idx_hbm, idx_v)

            def scan_body(v, cur):
                rv = idx_v[pl.ds(16 * v, 16)]
                jv = 16 * v + i16
                m = (rv >= base) & (rv < base + tile_r)
                cur = jnp.minimum(cur, CAP - 16)
                plsc.store_compressed(loc_r.at[pl.ds(cur, 16)], rv, mask=m)
                plsc.store_compressed(loc_j.at[pl.ds(cur, 16)], jv, mask=m)
                return cur + _popcount(m)

            nloc = jnp.int32(0)
            nlvec = (nloc + 15) // 16

            def chunk_lo(k):
                return jnp.minimum(base + 128 * k, last_full)

            def start_dma(k, buf, sem):
                for c1 in range(8):
                    pltpu.async_copy(
                        tab_hbm.at[pl.ds(8 * c1, 8), pl.ds(chunk_lo(k), 128)],
                        buf.at[pl.ds(8 * c1, 8)], sem)

            def wait_dma(k, buf, sem):
                for c1 in range(8):
                    pltpu.make_async_copy(
                        tab_hbm.at[pl.ds(8 * c1, 8), pl.ds(chunk_lo(k), 128)],
                        buf.at[pl.ds(8 * c1, 8)], sem).wait()

            def process(buf, lo, width, cur_out):
                return cur_out

            def _unused(buf, lo, width, cur_out):
                def sel_body(v, cc):
                    rv = loc_r[pl.ds(16 * v, 16)]
                    jv = loc_j[pl.ds(16 * v, 16)]
                    m = (rv >= lo) & (rv < lo + width)
                    cc = jnp.minimum(cc, CCAP - 16)
                    plsc.store_compressed(ch_r.at[pl.ds(cc, 16)], rv, mask=m)
                    plsc.store_compressed(ch_j.at[pl.ds(cc, 16)], jv, mask=m)
                    return cc + _popcount(m)

                nh = lax.fori_loop(0, nlvec, sel_body, jnp.int32(0))

                def row_body(i, cur):
                    w0 = (i // 16) * 16
                    li = i - w0
                    rsc = _extract(ch_r[pl.ds(w0, 16)], li)
                    jsc = _extract(ch_j[pl.ds(w0, 16)], li)
                    lane = jnp.broadcast_to(rsc - lo, (16,))
                    pos = jnp.minimum(cur, CAP - 1)
                    for q in range(4):
                        vals = plsc.load_gather(buf, [i16 + 16 * q, lane])
                        st_rows[pos, pl.ds(16 * q, 16)] = vals
                    plsc.store_scatter(st_j, [jnp.broadcast_to(pos, (16,))],
                                       jnp.broadcast_to(jsc, (16,)),
                                       mask=i16 == 0)
                    return cur + 1

                return lax.fori_loop(0, nh, row_body, cur_out)

            cp0 = start_dma(0, buf0, sem0)
            cp1 = start_dma(1, buf1, sem1)
            del cp0, cp1

            def pair_body(g, cur_out):
                k0 = 2 * g
                wait_dma(k0, buf0, sem0)
                cur_out = process(buf0, chunk_lo(k0), 128, cur_out)
                start_dma(k0 + 2, buf0, sem0)
                wait_dma(k0 + 1, buf1, sem1)
                cur_out = process(buf1, chunk_lo(k0 + 1), 128, cur_out)
                start_dma(k0 + 3, buf1, sem1)
                return cur_out

            npair = ncols // 2
            cur_out = lax.fori_loop(0, npair, pair_body, jnp.int32(0))
            wait_dma(2 * npair, buf0, sem0)
            wait_dma(2 * npair + 1, buf1, sem1)
            if ncols % 2:
                cur_out = process(buf0, chunk_lo(2 * npair), 128, cur_out)
            tail_dyn = jnp.int32(tail) + jnp.where(wid >= NW, 128, 0)
            for c1 in range(8):
                pltpu.sync_copy(
                    tab_hbm.at[pl.ds(8 * c1, 8), pl.ds(tail_dyn, 128)],
                    buf1.at[pl.ds(8 * c1, 8)])
            cur_out = process(buf1, jnp.int32(tail), tail_w, cur_out)

            for c in range(NSC):
                for t in range(4):
                    jj[c, pl.ds(16 * t, 16)] = st_j[pl.ds(64 * c + 16 * t, 16)]
            copies = [
                pltpu.async_copy(st_rows.at[pl.ds(64 * c, 64)],
                                 out_hbm.at[jj.at[c]], sem_s)
                for c in range(NSC)
            ]
            for cp in copies:
                cp.wait()

        run_table(users_hbm, ut_hbm, uout_hbm, TILE_R_U, COLS_U, TAIL_U,
                  TAIL_W_U)
        run_table(movies_hbm, mt_hbm, mout_hbm, TILE_R_M, COLS_M, TAIL_M,
                  TAIL_W_M)

    return gather


MBLK = 2048


def _r16(x):
    return x.astype(jnp.bfloat16).astype(jnp.float32)


def _mlp_body(ue, me, w1a, w1b, b1, w2, b2, w3, b3, out):
    h = jnp.dot(_r16(ue[:, :D]), w1a[...], preferred_element_type=jnp.float32,
                precision=lax.Precision.HIGHEST)
    h = h + jnp.dot(_r16(me[:, :D]), w1b[...],
                    preferred_element_type=jnp.float32,
                    precision=lax.Precision.HIGHEST)
    h = _r16(jnp.maximum(h + b1[...], 0.0))
    h = _r16(jnp.maximum(
        jnp.dot(h, w2[...], preferred_element_type=jnp.float32,
                precision=lax.Precision.HIGHEST) + b2[...], 0.0))
    out[...] = jnp.maximum(jnp.sum(h * w3[...], axis=1) + b3[0, 0], 0.0)


def kernel(users, movies, user_table, movie_table, W1, b1, W2, b2, W3, b3):
    uo, mo = _build_gather()(users.astype(jnp.int32),
                             movies.astype(jnp.int32),
                             user_table.T, movie_table.T)
    out = pl.pallas_call(
        _mlp_body,
        grid=(B // MBLK,),
        in_specs=[
            pl.BlockSpec((MBLK, 128), lambda i: (i, 0)),
            pl.BlockSpec((MBLK, 128), lambda i: (i, 0)),
            pl.BlockSpec((D, 64), lambda i: (0, 0)),
            pl.BlockSpec((D, 64), lambda i: (0, 0)),
            pl.BlockSpec((1, 64), lambda i: (0, 0)),
            pl.BlockSpec((64, 16), lambda i: (0, 0)),
            pl.BlockSpec((1, 16), lambda i: (0, 0)),
            pl.BlockSpec((1, 16), lambda i: (0, 0)),
            pl.BlockSpec((1, 1), lambda i: (0, 0)),
        ],
        out_specs=pl.BlockSpec((MBLK,), lambda i: (i,)),
        out_shape=jax.ShapeDtypeStruct((B,), jnp.float32),
    )(uo, mo, W1[:D], W1[D:], b1.reshape(1, 64), W2, b2.reshape(1, 16),
      W3.reshape(1, 16), b3.reshape(1, 1))
    return out


# speedup vs baseline: 2.8415x; 2.8370x over previous
import functools

import jax
import jax.numpy as jnp
from jax import lax
from jax.experimental import pallas as pl
from jax.experimental.pallas import tpu as pltpu
from jax.experimental.pallas import tpu_sc as plsc

B = 16384
D = 64
NC = 2
NS = 16
NW = NC * NS

PHYS_U = 7813 * 128
PHYS_M = 782 * 128
TILE_R_U = 31360
TILE_R_M = 3200
W_U = 2048
W_M = 256
NWIN_U = 16
NWIN_M = 13
CAP = 704
WCAP = 96
IDXW = 2048
DUMMY = B
OUT_ROWS = B + 8
NSC = CAP // 64


def _iota16():
    return lax.iota(jnp.int32, 16)


def _scalar(vec):
    return lax.reduce_max(vec, axes=(0,))


def _extract(vec, lane):
    return _scalar(jnp.where(_iota16() == lane, vec, jnp.int32(-2**31)))


def _popcount(mask):
    return _scalar(plsc.all_reduce_population_count(mask))


@functools.lru_cache(maxsize=None)
def _build_gather():
    mesh = plsc.VectorSubcoreMesh(core_axis_name="c", subcore_axis_name="s",
                                  num_cores=NC)

    @functools.partial(
        pl.kernel,
        mesh=mesh,
        out_type=(
            jax.ShapeDtypeStruct((OUT_ROWS, 128), jnp.float32),
            jax.ShapeDtypeStruct((OUT_ROWS, 128), jnp.float32),
        ),
        scratch_types=[
            pltpu.VMEM((IDXW,), jnp.int32),
            pltpu.VMEM((8, W_U), jnp.float32),
            pltpu.VMEM((8, W_U), jnp.float32),
            pltpu.VMEM((CAP,), jnp.int32),
            pltpu.VMEM((CAP,), jnp.int32),
            pltpu.VMEM((WCAP,), jnp.int32),
            pltpu.VMEM((WCAP,), jnp.int32),
            pltpu.VMEM((CAP, 128), jnp.float32),
            pltpu.VMEM((CAP,), jnp.int32),
            pltpu.VMEM((NSC, 64), jnp.int32),
            pltpu.SemaphoreType.DMA,
            pltpu.SemaphoreType.DMA,
            pltpu.SemaphoreType.DMA,
        ],
        compiler_params=pltpu.CompilerParams(use_tc_tiling_on_sc=True,
                                             needs_layout_passes=False),
    )
    def gather(users_hbm, movies_hbm, ut_hbm, mt_hbm, uout_hbm, mout_hbm,
               idx_v, buf0, buf1, loc_r, loc_j, w_r, w_j, st_rows, st_j,
               jj, sem0, sem1, sem_s):
        wid = lax.axis_index("s") * NC + lax.axis_index("c")
        i16 = _iota16()
        sub8 = i16 % 8
        half = i16 < 8
        big = jnp.full((16,), 2**30, jnp.int32)
        dummy16 = jnp.full((16,), DUMMY, jnp.int32)

        def run_table(idx_hbm, tab_hbm, out_hbm, tile_r, win, nwin, rphys):
            base = wid * tile_r
            last_lo = rphys - win

            for v in range(CAP // 16):
                loc_r[pl.ds(16 * v, 16)] = big
                st_j[pl.ds(16 * v, 16)] = dummy16

            def round_body(rnd, cur):
                pltpu.sync_copy(idx_hbm.at[pl.ds(rnd * IDXW, IDXW)], idx_v)

                def scan_body(v, cur):
                    rv = idx_v[pl.ds(16 * v, 16)]
                    jv = rnd * IDXW + 16 * v + i16
                    m = (rv >= base) & (rv < base + tile_r)
                    cur = jnp.minimum(cur, CAP - 16)
                    plsc.store_compressed(loc_r.at[pl.ds(cur, 16)], rv,
                                          mask=m)
                    plsc.store_compressed(loc_j.at[pl.ds(cur, 16)], jv,
                                          mask=m)
                    return cur + _popcount(m)

                return lax.fori_loop(0, IDXW // 16, scan_body, cur)

            nloc = lax.fori_loop(0, B // IDXW, round_body, jnp.int32(0))
            nlvec = (nloc + 15) // 16

            def win_lo(w):
                return jnp.minimum(base + win * w, last_lo)

            def src_slice(t):
                w, c1 = t // 8, t % 8
                return tab_hbm.at[pl.ds(8 * c1, 8), pl.ds(win_lo(w), win)]

            def start_dma(t, buf, sem):
                return pltpu.async_copy(src_slice(t),
                                        buf.at[:, pl.ds(0, win)], sem)

            def wait_dma(t, buf, sem):
                pltpu.make_async_copy(src_slice(t),
                                      buf.at[:, pl.ds(0, win)], sem).wait()

            def make_window_list(lo):
                def sel_body(v, cc):
                    rv = loc_r[pl.ds(16 * v, 16)]
                    jv = loc_j[pl.ds(16 * v, 16)]
                    m = (rv >= lo) & (rv < lo + win)
                    cc = jnp.minimum(cc, WCAP - 16)
                    plsc.store_compressed(w_r.at[pl.ds(cc, 16)], rv, mask=m)
                    plsc.store_compressed(w_j.at[pl.ds(cc, 16)], jv, mask=m)
                    return cc + _popcount(m)

                return lax.fori_loop(0, nlvec, sel_body, jnp.int32(0))

            def process_band(c1, buf, lo, nh, cur_out):
                def pair_body(p, _):
                    w0 = ((2 * p) // 16) * 16
                    l0 = 2 * p - w0
                    vec = w_r[pl.ds(w0, 16)]
                    r0 = _extract(vec, l0)
                    r1 = _extract(vec, l0 + 1)
                    pos0 = jnp.minimum(cur_out + 2 * p, CAP - 2)
                    lane = jnp.where(half, r0 - lo, r1 - lo)
                    lane = jnp.clip(lane, 0, win - 1)
                    vals = plsc.load_gather(buf, [sub8, lane])
                    outrow = jnp.where(half, pos0, pos0 + 1)
                    m = half | jnp.broadcast_to(2 * p + 1 < nh, (16,))
                    plsc.store_scatter(st_rows, [outrow, 8 * c1 + sub8],
                                       vals, mask=m)
                    return jnp.int32(0)

                lax.fori_loop(0, (nh + 1) // 2, pair_body, jnp.int32(0))

            def fill_st_j(nh, cur_out):
                def jb(v, _):
                    jv = w_j[pl.ds(16 * v, 16)]
                    tgt = jnp.minimum(cur_out + 16 * v + i16, CAP - 1)
                    m = 16 * v + i16 < nh
                    plsc.store_scatter(st_j, [tgt], jv, mask=m)
                    return jnp.int32(0)

                lax.fori_loop(0, (nh + 15) // 16, jb, jnp.int32(0))

            start_dma(0, buf0, sem0)
            start_dma(1, buf1, sem1)
            bufs = (buf0, buf1)
            sems = (sem0, sem1)

            def window_body(w, cur_out):
                lo = win_lo(w)
                nh = make_window_list(lo)
                for c1 in range(8):
                    t = 8 * w + c1
                    buf, sem = bufs[c1 % 2], sems[c1 % 2]
                    wait_dma(t, buf, sem)
                    process_band(c1, buf, lo, nh, cur_out)
                    start_dma(t + 2, buf, sem)
                fill_st_j(nh, cur_out)
                return cur_out + nh

            lax.fori_loop(0, nwin, window_body, jnp.int32(0))
            wait_dma(8 * nwin, buf0, sem0)
            wait_dma(8 * nwin + 1, buf1, sem1)

            for c in range(NSC):
                for t in range(4):
                    jj[c, pl.ds(16 * t, 16)] = st_j[pl.ds(64 * c + 16 * t, 16)]
            copies = [
                pltpu.async_copy(st_rows.at[pl.ds(64 * c, 64)],
                                 out_hbm.at[jj.at[c]], sem_s)
                for c in range(NSC)
            ]
            for cp in copies:
                cp.wait()

        run_table(users_hbm, ut_hbm, uout_hbm, TILE_R_U, W_U, NWIN_U, PHYS_U)
        run_table(movies_hbm, mt_hbm, mout_hbm, TILE_R_M, W_M, NWIN_M, PHYS_M)

    return gather


MBLK = 2048


def _r16(x):
    return x.astype(jnp.bfloat16).astype(jnp.float32)


def _b16(x):
    return x.astype(jnp.bfloat16)


def _mlp_body(ue, me, w1a, w1b, b1, w2, b2, w3, b3, out):
    h = jnp.dot(_b16(ue[:, :D]), _b16(w1a[...]),
                preferred_element_type=jnp.float32)
    h = h + jnp.dot(_b16(me[:, :D]), _b16(w1b[...]),
                    preferred_element_type=jnp.float32)
    h = _b16(jnp.maximum(h + b1[...], 0.0))
    h = jnp.dot(h, _b16(w2[...]), preferred_element_type=jnp.float32)
    h = _b16(jnp.maximum(h + b2[...], 0.0))
    hs = jnp.sum(_r16(h) * _r16(w3[...]), axis=1)
    out[...] = jnp.maximum(hs + b3[0, 0], 0.0)


def kernel(users, movies, user_table, movie_table, W1, b1, W2, b2, W3, b3):
    uo, mo = _build_gather()(users.astype(jnp.int32),
                             movies.astype(jnp.int32),
                             user_table.T, movie_table.T)
    out = pl.pallas_call(
        _mlp_body,
        grid=(B // MBLK,),
        in_specs=[
            pl.BlockSpec((MBLK, 128), lambda i: (i, 0)),
            pl.BlockSpec((MBLK, 128), lambda i: (i, 0)),
            pl.BlockSpec((D, 64), lambda i: (0, 0)),
            pl.BlockSpec((D, 64), lambda i: (0, 0)),
            pl.BlockSpec((1, 64), lambda i: (0, 0)),
            pl.BlockSpec((64, 16), lambda i: (0, 0)),
            pl.BlockSpec((1, 16), lambda i: (0, 0)),
            pl.BlockSpec((1, 16), lambda i: (0, 0)),
            pl.BlockSpec((1, 1), lambda i: (0, 0)),
        ],
        out_specs=pl.BlockSpec((MBLK,), lambda i: (i,)),
        out_shape=jax.ShapeDtypeStruct((B,), jnp.float32),
    )(uo, mo, W1[:D], W1[D:], b1.reshape(1, 64), W2, b2.reshape(1, 16),
      W3.reshape(1, 16), b3.reshape(1, 1))
    return out
